# baseline (speedup 1.0000x reference)
import jax
import jax.numpy as jnp
from jax.experimental import pallas as pl
from jax.experimental.pallas import tpu as pltpu

_CHUNKS = (1024, 3072, 4096)
_READS_AHEAD = 2


def _copy_body(pe_ref, out_ref, buf, *sems):
    n = len(_CHUNKS)
    isems = sems[:n]
    osems = sems[n:]
    offs = []
    o = 0
    for c in _CHUNKS:
        offs.append(o)
        o += c

    def read(i):
        cp = pltpu.make_async_copy(
            pe_ref.at[pl.ds(offs[i], _CHUNKS[i])],
            buf.at[pl.ds(offs[i], _CHUNKS[i])],
            isems[i],
        )
        cp.start()
        return cp

    def write(i):
        cp = pltpu.make_async_copy(
            buf.at[pl.ds(offs[i], _CHUNKS[i])],
            out_ref.at[pl.ds(offs[i], _CHUNKS[i])],
            osems[i],
        )
        cp.start()
        return cp

    ins = {i: read(i) for i in range(_READS_AHEAD)}
    outs = {}
    for i in range(n):
        ins[i].wait()
        outs[i] = write(i)
        if i + _READS_AHEAD < n:
            ins[i + _READS_AHEAD] = read(i + _READS_AHEAD)
    for i in range(n):
        outs[i].wait()


def kernel(x, pe):
    seq_len = x.shape[1]
    hidden = pe.shape[1]
    out = pl.pallas_call(
        _copy_body,
        in_specs=[pl.BlockSpec(memory_space=pl.ANY)],
        out_specs=pl.BlockSpec(memory_space=pl.ANY),
        out_shape=jax.ShapeDtypeStruct((seq_len, hidden), pe.dtype),
        scratch_shapes=(
            [pltpu.VMEM((seq_len, hidden), pe.dtype)]
            + [pltpu.SemaphoreType.DMA] * (2 * len(_CHUNKS))
        ),
    )(pe)
    return out[None, ...]


# speedup vs baseline: 1.0570x; 1.0570x over previous
import jax
import jax.numpy as jnp
from jax.experimental import pallas as pl
from jax.experimental.pallas import tpu as pltpu


def _make_copy_body(chunks):
    offs = []
    o = 0
    for c in chunks:
        offs.append(o)
        o += c
    n = len(chunks)

    def copy_body(pe_ref, out_ref, buf, *sems):
        isems = sems[:n]
        osems = sems[n:]

        def read(i):
            cp = pltpu.make_async_copy(
                pe_ref.at[pl.ds(offs[i], chunks[i])],
                buf.at[pl.ds(offs[i], chunks[i])],
                isems[i],
            )
            cp.start()
            return cp

        def write(i):
            cp = pltpu.make_async_copy(
                buf.at[pl.ds(offs[i], chunks[i])],
                out_ref.at[pl.ds(offs[i], chunks[i])],
                osems[i],
            )
            cp.start()
            return cp

        ins = [read(i) for i in range(n)]
        outs = []
        for i in range(n):
            ins[i].wait()
            outs.append(write(i))
        for cp in outs:
            cp.wait()

    return copy_body


def kernel(x, pe):
    seq_len = x.shape[1]
    hidden = pe.shape[1]
    half = seq_len // 2
    chunks = (half, seq_len - half)
    out = pl.pallas_call(
        _make_copy_body(chunks),
        in_specs=[pl.BlockSpec(memory_space=pl.ANY)],
        out_specs=pl.BlockSpec(memory_space=pl.ANY),
        out_shape=jax.ShapeDtypeStruct((seq_len, hidden), pe.dtype),
        scratch_shapes=(
            [pltpu.VMEM((seq_len, hidden), pe.dtype)]
            + [pltpu.SemaphoreType.DMA] * (2 * len(chunks))
        ),
    )(pe)
    return out[None, ...]
